# baseline (speedup 1.0000x reference)
import functools

import jax
import jax.numpy as jnp
from jax import lax
from jax.experimental import pallas as pl
from jax.experimental.pallas import tpu as pltpu
from jax.experimental.pallas import tpu_sc as plsc

B = 16384
F = 26
V = 100000
E = 16
H = 32
FE = F * E
NROWS = B * F
EPS = 1e-5

NC = 2
NS = 16
NW = NC * NS
N_PER_W = NROWS // NW
IDX_W = 128
CH = 1024
DESC = CH // IDX_W
NCH = N_PER_W // CH

_HIGH = lax.Precision.HIGHEST


def _dot(x, y):
    return lax.dot_general(x, y, (((1,), (0,)), ((), ())), precision=_HIGH)


def _sc_gather(w2flat, w1flat, idx2d):
    mesh = plsc.VectorSubcoreMesh(core_axis_name="c", subcore_axis_name="s")

    @functools.partial(
        pl.kernel,
        out_type=[
            jax.ShapeDtypeStruct((NROWS, E), jnp.float32),
            jax.ShapeDtypeStruct((NROWS,), jnp.float32),
        ],
        mesh=mesh,
        compiler_params=pltpu.CompilerParams(use_tc_tiling_on_sc=False),
        scratch_types=[
            pltpu.VMEM((N_PER_W // IDX_W, IDX_W), jnp.int32),
            pltpu.VMEM((CH, E), jnp.float32),
            pltpu.VMEM((CH,), jnp.float32),
            pltpu.SemaphoreType.DMA,
            pltpu.SemaphoreType.DMA,
        ],
    )
    def k(w2_hbm, w1_hbm, idx_hbm, rows_out, w1_out, idx_v, rows_v, w1_v, sem2, sem1):
        wid = lax.axis_index("s") * NC + lax.axis_index("c")
        row0 = wid * (N_PER_W // IDX_W)
        base = wid * N_PER_W
        pltpu.sync_copy(idx_hbm.at[pl.ds(row0, N_PER_W // IDX_W)], idx_v)

        def chunk(ch, _):
            cps = []
            for j in range(DESC):
                cps.append(pltpu.async_copy(
                    w2_hbm.at[idx_v.at[ch * DESC + j]],
                    rows_v.at[pl.ds(j * IDX_W, IDX_W)], sem2))
                cps.append(pltpu.async_copy(
                    w1_hbm.at[idx_v.at[ch * DESC + j]],
                    w1_v.at[pl.ds(j * IDX_W, IDX_W)], sem1))
            for cp in cps:
                cp.wait()
            off = base + ch * CH
            pltpu.sync_copy(rows_v, rows_out.at[pl.ds(off, CH)])
            pltpu.sync_copy(w1_v, w1_out.at[pl.ds(off, CH)])
            return _

        lax.fori_loop(0, NCH, chunk, 0, unroll=False)

    return k(w2flat, w1flat, idx2d)


def _pass_a_body(rows_ref, w1v_ref, xv_ref, wl1t_ref, bl1_ref, r_ref, s_ref,
                 h1_ref, part_ref, gram_ref, srow_ref):
    x = xv_ref[...]
    xe = _dot(x, r_ref[...])
    e2 = rows_ref[...] * xe
    h1 = _dot(e2, wl1t_ref[...]) + bl1_ref[...]
    h1_ref[...] = h1
    s = _dot(e2, s_ref[...])
    fm2 = 0.5 * (jnp.sum(s * s, 1, keepdims=True)
                 - jnp.sum(e2 * e2, 1, keepdims=True))
    fm1 = jnp.sum(w1v_ref[...] * x, 1, keepdims=True)
    part_ref[...] = fm1 + fm2
    g = lax.dot_general(h1, h1, (((0,), (0,)), ((), ())), precision=_HIGH)
    sr = jnp.sum(h1, 0, keepdims=True)
    i = pl.program_id(0)

    @pl.when(i == 0)
    def _():
        gram_ref[...] = g
        srow_ref[...] = sr

    @pl.when(i != 0)
    def _():
        gram_ref[...] += g
        srow_ref[...] += sr


def _pass_a(rows2d, w1v2d, xv, wl1t, bl1r, rmat, smat, bb=512, interpret=False):
    nb = B // bb
    return pl.pallas_call(
        _pass_a_body,
        grid=(nb,),
        in_specs=[
            pl.BlockSpec((bb, FE), lambda i: (i, 0)),
            pl.BlockSpec((bb, F), lambda i: (i, 0)),
            pl.BlockSpec((bb, F), lambda i: (i, 0)),
            pl.BlockSpec((FE, H), lambda i: (0, 0)),
            pl.BlockSpec((1, H), lambda i: (0, 0)),
            pl.BlockSpec((F, FE), lambda i: (0, 0)),
            pl.BlockSpec((FE, E), lambda i: (0, 0)),
        ],
        out_specs=[
            pl.BlockSpec((bb, H), lambda i: (i, 0)),
            pl.BlockSpec((bb, 1), lambda i: (i, 0)),
            pl.BlockSpec((H, H), lambda i: (0, 0)),
            pl.BlockSpec((1, H), lambda i: (0, 0)),
        ],
        out_shape=[
            jax.ShapeDtypeStruct((B, H), jnp.float32),
            jax.ShapeDtypeStruct((B, 1), jnp.float32),
            jax.ShapeDtypeStruct((H, H), jnp.float32),
            jax.ShapeDtypeStruct((1, H), jnp.float32),
        ],
        interpret=interpret,
    )(rows2d, w1v2d, xv, wl1t, bl1r, rmat, smat)


def _pass_b_body(h1_ref, part_ref, gram_ref, srow_ref, wl2_ref, wl2t_ref,
                 eye_ref, g1_ref, bt1_ref, g2_ref, bt2_ref, bl2_ref, bias_ref,
                 out_ref):
    binv = 1.0 / B
    eye = eye_ref[...]
    m1 = srow_ref[...] * binv
    gram_n = gram_ref[...] * binv
    diag_m1 = eye * m1
    outer = _dot(diag_m1, jnp.broadcast_to(m1, (H, H)))
    cov1 = gram_n - outer
    v1 = jnp.sum(cov1 * eye, 0, keepdims=True)
    c = g1_ref[...] * lax.rsqrt(v1 + EPS)
    d = bt1_ref[...] - c * m1
    covn = _dot(eye * c, cov1 * c)
    tt = _dot(covn, wl2t_ref[...])
    v2 = jnp.sum(wl2t_ref[...] * tt, 0, keepdims=True)
    a = g2_ref[...] * lax.rsqrt(v2 + EPS)
    u = _dot(a, wl2_ref[...])
    m2 = _dot(bt1_ref[...], wl2t_ref[...]) + bl2_ref[...]
    k = (jnp.sum(u * d, 1, keepdims=True)
         + jnp.sum(a * bl2_ref[...], 1, keepdims=True)
         + jnp.sum(bt2_ref[...] - a * m2, 1, keepdims=True)
         + bias_ref[...])
    uc = u * c
    out_ref[...] = (part_ref[...]
                    + jnp.sum(h1_ref[...] * uc, 1, keepdims=True) + k)


def _pass_b(h1, part, gram, srow, wl2, wl2t, eye, g1r, bt1r, g2r, bt2r, bl2r,
            biasr, bb=512, interpret=False):
    nb = B // bb
    const = lambda i: (0, 0)
    return pl.pallas_call(
        _pass_b_body,
        grid=(nb,),
        in_specs=[
            pl.BlockSpec((bb, H), lambda i: (i, 0)),
            pl.BlockSpec((bb, 1), lambda i: (i, 0)),
            pl.BlockSpec((H, H), const),
            pl.BlockSpec((1, H), const),
            pl.BlockSpec((H, H), const),
            pl.BlockSpec((H, H), const),
            pl.BlockSpec((H, H), const),
            pl.BlockSpec((1, H), const),
            pl.BlockSpec((1, H), const),
            pl.BlockSpec((1, H), const),
            pl.BlockSpec((1, H), const),
            pl.BlockSpec((1, H), const),
            pl.BlockSpec((1, 1), const),
        ],
        out_specs=pl.BlockSpec((bb, 1), lambda i: (i, 0)),
        out_shape=jax.ShapeDtypeStruct((B, 1), jnp.float32),
        interpret=interpret,
    )(h1, part, gram, srow, wl2, wl2t, eye, g1r, bt1r, g2r, bt2r, bl2r, biasr)


def kernel(Xi, Xv, W1, W2, Wl1, bl1, g1, bt1, Wl2, bl2, g2, bt2, bias):
    idx = Xi[:, :, 0].astype(jnp.int32) + jnp.arange(F, dtype=jnp.int32)[None, :] * V
    idx2d = idx.reshape(NROWS // IDX_W, IDX_W)
    w2flat = W2.reshape(F * V, E)
    w1flat = W1.reshape(F * V)

    rows, w1v = _sc_gather(w2flat, w1flat, idx2d)
    rows2d = rows.reshape(B, FE)
    w1v2d = w1v.reshape(B, F)

    eyeF = jnp.eye(F, dtype=jnp.float32)
    eyeE = jnp.eye(E, dtype=jnp.float32)
    rmat = jnp.kron(eyeF, jnp.ones((1, E), jnp.float32))
    smat = jnp.kron(jnp.ones((F, 1), jnp.float32), eyeE)
    wl1t = Wl1.T
    bl1r = bl1.reshape(1, H)

    h1, part, gram, srow = _pass_a(rows2d, w1v2d, Xv, wl1t, bl1r, rmat, smat)

    out2d = _pass_b(h1, part, gram, srow, Wl2, Wl2.T,
                    jnp.eye(H, dtype=jnp.float32),
                    g1.reshape(1, H), bt1.reshape(1, H), g2.reshape(1, H),
                    bt2.reshape(1, H), bl2.reshape(1, H), bias.reshape(1, 1))
    return out2d.reshape(B)


# speedup vs baseline: 4.1775x; 4.1775x over previous
import functools

import jax
import jax.numpy as jnp
from jax import lax
from jax.experimental import pallas as pl
from jax.experimental.pallas import tpu as pltpu
from jax.experimental.pallas import tpu_sc as plsc

B = 16384
F = 26
V = 100000
E = 16
H = 32
FE = F * E
EPS = 1e-5

NC = 2
NS = 16
NW = NC * NS
PW = FE // NW
CB = 4096
NCB = B // CB
L = 16

_HIGH = lax.Precision.HIGHEST


def _dot(x, y):
    return lax.dot_general(x, y, (((1,), (0,)), ((), ())), precision=_HIGH)


def _sc_gather(w2t, w1t, idxt):
    mesh = plsc.VectorSubcoreMesh(core_axis_name="c", subcore_axis_name="s")

    @functools.partial(
        pl.kernel,
        out_type=[
            jax.ShapeDtypeStruct((FE, B), jnp.float32),
            jax.ShapeDtypeStruct((F, B), jnp.float32),
        ],
        mesh=mesh,
        compiler_params=pltpu.CompilerParams(needs_layout_passes=False),
        scratch_types=[
            pltpu.VMEM((V,), jnp.float32),
            pltpu.VMEM((CB,), jnp.int32),
            pltpu.VMEM((CB,), jnp.float32),
        ],
    )
    def k(w2_hbm, w1_hbm, idx_hbm, e2t_out, w1v_out, plane_v, idx_v, out_v):
        wid = lax.axis_index("s") * NC + lax.axis_index("c")

        def do_plane(table_hbm, p, f, out_hbm):
            pltpu.sync_copy(table_hbm.at[p], plane_v)
            for c in range(NCB):
                pltpu.sync_copy(idx_hbm.at[f, pl.ds(c * CB, CB)], idx_v)

                def gat(j, carry):
                    vidx = idx_v[pl.ds(j * L, L)]
                    out_v[pl.ds(j * L, L)] = plsc.load_gather(plane_v, [vidx])
                    return carry

                lax.fori_loop(0, CB // L, gat, 0)
                pltpu.sync_copy(out_v, out_hbm.at[p, pl.ds(c * CB, CB)])

        def plane_loop(i, carry):
            p = wid * PW + i
            do_plane(w2_hbm, p, p // E, e2t_out)
            return carry

        lax.fori_loop(0, PW, plane_loop, 0)

        @pl.when(wid < F)
        def _():
            do_plane(w1_hbm, wid, wid, w1v_out)

    return k(w2t, w1t, idxt)


def _pass_a_body(e2t_ref, w1v_ref, xv_ref, wl1_ref, bl1_ref, r_ref, s_ref,
                 h1_ref, part_ref, gram_ref, scol_ref):
    xvt = xv_ref[...]
    xe = _dot(r_ref[...], xvt)
    e2 = e2t_ref[...] * xe
    h1 = _dot(wl1_ref[...], e2) + bl1_ref[...]
    h1_ref[...] = h1
    st = _dot(s_ref[...], e2)
    fm2 = 0.5 * (jnp.sum(st * st, 0, keepdims=True)
                 - jnp.sum(e2 * e2, 0, keepdims=True))
    fm1 = jnp.sum(w1v_ref[...] * xvt, 0, keepdims=True)
    part_ref[...] = fm1 + fm2
    g = lax.dot_general(h1, h1, (((1,), (1,)), ((), ())), precision=_HIGH)
    sc = jnp.sum(h1, 1, keepdims=True)
    i = pl.program_id(0)

    @pl.when(i == 0)
    def _():
        gram_ref[...] = g
        scol_ref[...] = sc

    @pl.when(i != 0)
    def _():
        gram_ref[...] += g
        scol_ref[...] += sc


def _pass_a(e2t, w1vt, xvt, wl1, bl1c, rt, st, bb=2048, interpret=False):
    nb = B // bb
    const = lambda i: (0, 0)
    return pl.pallas_call(
        _pass_a_body,
        grid=(nb,),
        in_specs=[
            pl.BlockSpec((FE, bb), lambda i: (0, i)),
            pl.BlockSpec((F, bb), lambda i: (0, i)),
            pl.BlockSpec((F, bb), lambda i: (0, i)),
            pl.BlockSpec((H, FE), const),
            pl.BlockSpec((H, 1), const),
            pl.BlockSpec((FE, F), const),
            pl.BlockSpec((E, FE), const),
        ],
        out_specs=[
            pl.BlockSpec((H, bb), lambda i: (0, i)),
            pl.BlockSpec((1, bb), lambda i: (0, i)),
            pl.BlockSpec((H, H), const),
            pl.BlockSpec((H, 1), const),
        ],
        out_shape=[
            jax.ShapeDtypeStruct((H, B), jnp.float32),
            jax.ShapeDtypeStruct((1, B), jnp.float32),
            jax.ShapeDtypeStruct((H, H), jnp.float32),
            jax.ShapeDtypeStruct((H, 1), jnp.float32),
        ],
        interpret=interpret,
    )(e2t, w1vt, xvt, wl1, bl1c, rt, st)


def _pass_b_body(h1_ref, part_ref, gram_ref, scol_ref, wl2_ref, eye_ref,
                 g1_ref, bt1_ref, g2_ref, bt2_ref, bl2_ref, bias_ref,
                 out_ref):
    binv = 1.0 / B
    eye = eye_ref[...]
    wl2 = wl2_ref[...]
    m1 = scol_ref[...] * binv
    outer = lax.dot_general(m1, m1, (((1,), (1,)), ((), ())), precision=_HIGH)
    cov1 = gram_ref[...] * binv - outer
    v1 = jnp.sum(cov1 * eye, 1, keepdims=True)
    c = g1_ref[...] * lax.rsqrt(v1 + EPS)
    c_row = jnp.sum(eye * c, 0, keepdims=True)
    covn = (c * cov1) * c_row
    t = _dot(wl2, covn)
    v2 = jnp.sum(t * wl2, 1, keepdims=True)
    a = g2_ref[...] * lax.rsqrt(v2 + EPS)
    u = lax.dot_general(wl2, a, (((0,), (0,)), ((), ())), precision=_HIGH)
    m2 = _dot(wl2, bt1_ref[...]) + bl2_ref[...]
    d = bt1_ref[...] - c * m1
    k = (jnp.sum(u * d) + jnp.sum(a * bl2_ref[...])
         + jnp.sum(bt2_ref[...] - a * m2) + bias_ref[0, 0])
    out_ref[...] = (part_ref[...] + k
                    + lax.dot_general(u * c, h1_ref[...],
                                      (((0,), (0,)), ((), ())),
                                      precision=_HIGH))


def _pass_b(h1t, part, gram, scol, wl2, eye, g1c, bt1c, g2c, bt2c, bl2c,
            biasr, bb=2048, interpret=False):
    nb = B // bb
    const = lambda i: (0, 0)
    return pl.pallas_call(
        _pass_b_body,
        grid=(nb,),
        in_specs=[
            pl.BlockSpec((H, bb), lambda i: (0, i)),
            pl.BlockSpec((1, bb), lambda i: (0, i)),
            pl.BlockSpec((H, H), const),
            pl.BlockSpec((H, 1), const),
            pl.BlockSpec((H, H), const),
            pl.BlockSpec((H, H), const),
            pl.BlockSpec((H, 1), const),
            pl.BlockSpec((H, 1), const),
            pl.BlockSpec((H, 1), const),
            pl.BlockSpec((H, 1), const),
            pl.BlockSpec((H, 1), const),
            pl.BlockSpec((1, 1), const),
        ],
        out_specs=pl.BlockSpec((1, bb), lambda i: (0, i)),
        out_shape=jax.ShapeDtypeStruct((1, B), jnp.float32),
        interpret=interpret,
    )(h1t, part, gram, scol, wl2, eye, g1c, bt1c, g2c, bt2c, bl2c, biasr)


def kernel(Xi, Xv, W1, W2, Wl1, bl1, g1, bt1, Wl2, bl2, g2, bt2, bias):
    w2t = jnp.transpose(W2, (0, 2, 1)).reshape(FE, V)
    w1t = jnp.transpose(W1, (0, 2, 1)).reshape(F, V)
    idxt = Xi[:, :, 0].astype(jnp.int32).T
    xvt = Xv.T

    e2t, w1vt = _sc_gather(w2t, w1t, idxt)

    rt = jnp.kron(jnp.eye(F, dtype=jnp.float32), jnp.ones((E, 1), jnp.float32))
    st = jnp.kron(jnp.ones((1, F), jnp.float32), jnp.eye(E, dtype=jnp.float32))

    h1t, part, gram, scol = _pass_a(e2t, w1vt, xvt, Wl1, bl1.reshape(H, 1),
                                    rt, st)

    out = _pass_b(h1t, part, gram, scol, Wl2, jnp.eye(H, dtype=jnp.float32),
                  g1.reshape(H, 1), bt1.reshape(H, 1), g2.reshape(H, 1),
                  bt2.reshape(H, 1), bl2.reshape(H, 1), bias.reshape(1, 1))
    return out.reshape(B)
